# Initial kernel scaffold; baseline (speedup 1.0000x reference)
import jax
import jax.numpy as jnp
from jax.experimental import pallas as pl


def kernel(x, table, bias):
    raise NotImplementedError("write your pallas kernel here")


# speedup vs baseline: 1.2116x; 1.2116x over previous
import functools

import jax
import jax.numpy as jnp
from jax import lax
from jax.experimental import pallas as pl
from jax.experimental.pallas import tpu as pltpu
from jax.experimental.pallas import tpu_sc as plsc

BATCH = 4096
NUM_FIELDS = 26
NUM_FEATURES = 100000

NC = 2
NS = 16
L = 16
NW = NC * NS
CHUNK = BATCH // NW
GROUPS = CHUNK // L


@functools.partial(
    pl.kernel,
    out_type=jax.ShapeDtypeStruct((BATCH,), jnp.float32),
    mesh=plsc.VectorSubcoreMesh(core_axis_name="c", subcore_axis_name="s"),
    scratch_types=[
        pltpu.VMEM((NUM_FIELDS, CHUNK), jnp.int32),
        pltpu.VMEM((NUM_FIELDS, CHUNK), jnp.float32),
        pltpu.VMEM((L,), jnp.int32),
        pltpu.VMEM((L,), jnp.float32),
        pltpu.VMEM((CHUNK,), jnp.float32),
        pltpu.SemaphoreType.DMA,
        pltpu.SemaphoreType.DMA,
    ],
)
def _sc_linear(x_hbm, table_hbm, bias_hbm, out_hbm,
               idx_v, vals_v, zi_v, bias_v, out_v, gsem, bsem):
    wid = lax.axis_index("s") * NC + lax.axis_index("c")
    row_base = wid * CHUNK

    pltpu.sync_copy(x_hbm.at[wid], idx_v)

    zi_v[...] = jnp.zeros((L,), jnp.int32)
    bias_cp = pltpu.async_copy(bias_hbm.at[zi_v], bias_v, bsem)

    copies = [
        pltpu.async_copy(table_hbm.at[idx_v.at[f]], vals_v.at[f], gsem)
        for f in range(NUM_FIELDS)
    ]
    bias_cp.wait()
    for c in copies:
        c.wait()

    bias_vec = bias_v[...]
    for g in range(GROUPS):
        acc = bias_vec
        for f in range(NUM_FIELDS):
            acc = acc + vals_v[f, pl.ds(g * L, L)]
        out_v[pl.ds(g * L, L)] = acc

    pltpu.sync_copy(out_v, out_hbm.at[pl.ds(row_base, CHUNK)])


def kernel(x, table, bias):
    x_perm = x.reshape(NW, CHUNK, NUM_FIELDS).transpose(0, 2, 1)
    table_flat = table.reshape(NUM_FEATURES)
    return _sc_linear(x_perm, table_flat, bias)


# speedup vs baseline: 1.2198x; 1.0068x over previous
import functools

import jax
import jax.numpy as jnp
from jax import lax
from jax.experimental import pallas as pl
from jax.experimental.pallas import tpu as pltpu
from jax.experimental.pallas import tpu_sc as plsc

BATCH = 4096
NUM_FIELDS = 26
NUM_FEATURES = 100000

NC = 2
NS = 16
L = 16
NW = NC * NS
CHUNK = BATCH // NW
GROUPS = CHUNK // L


@functools.partial(
    pl.kernel,
    out_type=jax.ShapeDtypeStruct((BATCH,), jnp.float32),
    mesh=plsc.VectorSubcoreMesh(core_axis_name="c", subcore_axis_name="s"),
    scratch_types=[
        pltpu.VMEM((NUM_FIELDS, CHUNK), jnp.int32),
        pltpu.VMEM((NUM_FIELDS, CHUNK), jnp.float32),
        pltpu.VMEM((L,), jnp.int32),
        pltpu.VMEM((L,), jnp.float32),
        pltpu.VMEM((CHUNK,), jnp.float32),
        pltpu.SemaphoreType.DMA,
        pltpu.SemaphoreType.DMA,
    ],
)
def _sc_linear(x_hbm, table_hbm, bias_hbm, out_hbm,
               idx_v, vals_v, zi_v, bias_v, out_v, gsem, bsem):
    wid = lax.axis_index("s") * NC + lax.axis_index("c")
    row_base = wid * CHUNK

    pltpu.sync_copy(x_hbm.at[wid], idx_v)

    zi_v[...] = jnp.zeros((L,), jnp.int32)
    bias_cp = pltpu.async_copy(bias_hbm.at[zi_v], bias_v, bsem)

    def fire(f, carry):
        pltpu.async_copy(table_hbm.at[idx_v.at[f]], vals_v.at[f], gsem)
        return carry

    def drain(f, carry):
        pltpu.make_async_copy(table_hbm.at[idx_v.at[f]], vals_v.at[f], gsem).wait()
        return carry

    lax.fori_loop(0, NUM_FIELDS, fire, 0)
    bias_cp.wait()
    lax.fori_loop(0, NUM_FIELDS, drain, 0)

    bias_vec = bias_v[...]

    def reduce_group(g, carry):
        def add_field(f, acc):
            return acc + vals_v[f, pl.ds(g * L, L)]

        acc = lax.fori_loop(0, NUM_FIELDS, add_field, bias_vec)
        out_v[pl.ds(g * L, L)] = acc
        return carry

    lax.fori_loop(0, GROUPS, reduce_group, 0)

    pltpu.sync_copy(out_v, out_hbm.at[pl.ds(row_base, CHUNK)])


def kernel(x, table, bias):
    x_perm = x.reshape(NW, CHUNK, NUM_FIELDS).transpose(0, 2, 1)
    table_flat = table[:, 0]
    return _sc_linear(x_perm, table_flat, bias)


# speedup vs baseline: 1.3860x; 1.1362x over previous
import functools

import jax
import jax.numpy as jnp
from jax import lax
from jax.experimental import pallas as pl
from jax.experimental.pallas import tpu as pltpu
from jax.experimental.pallas import tpu_sc as plsc

BATCH = 4096
NUM_FIELDS = 26
NUM_FEATURES = 100000

NC = 2
NS = 16
L = 16
NW = NC * NS
CHUNK = BATCH // NW
GROUPS = CHUNK // L


@functools.partial(
    pl.kernel,
    out_type=jax.ShapeDtypeStruct((BATCH,), jnp.float32),
    mesh=plsc.VectorSubcoreMesh(core_axis_name="c", subcore_axis_name="s"),
    scratch_types=[
        pltpu.VMEM((NUM_FIELDS, CHUNK), jnp.int32),
        pltpu.VMEM((NUM_FIELDS, CHUNK), jnp.float32),
        pltpu.VMEM((L,), jnp.int32),
        pltpu.VMEM((L,), jnp.float32),
        pltpu.VMEM((CHUNK,), jnp.float32),
        pltpu.VMEM_SHARED((NUM_FEATURES,), jnp.float32),
        pltpu.SemaphoreType.DMA,
        pltpu.SemaphoreType.DMA,
    ],
)
def _sc_linear(x_hbm, table_hbm, bias_hbm, out_hbm,
               idx_v, vals_v, zi_v, bias_v, out_v, tbl_sh, gsem, bsem):
    wid = lax.axis_index("s") * NC + lax.axis_index("c")
    sid = lax.axis_index("s")
    row_base = wid * CHUNK

    @pl.when(sid == 0)
    def _():
        pltpu.sync_copy(table_hbm, tbl_sh)

    pltpu.sync_copy(x_hbm.at[wid], idx_v)

    zi_v[...] = jnp.zeros((L,), jnp.int32)
    bias_cp = pltpu.async_copy(bias_hbm.at[zi_v], bias_v, bsem)

    plsc.subcore_barrier()

    def fire(f, carry):
        pltpu.async_copy(tbl_sh.at[idx_v.at[f]], vals_v.at[f], gsem)
        return carry

    def drain(f, carry):
        pltpu.make_async_copy(tbl_sh.at[idx_v.at[f]], vals_v.at[f], gsem).wait()
        return carry

    lax.fori_loop(0, NUM_FIELDS, fire, 0)
    bias_cp.wait()
    lax.fori_loop(0, NUM_FIELDS, drain, 0)

    bias_vec = bias_v[...]

    def reduce_group(g, carry):
        def add_field(f, acc):
            return acc + vals_v[f, pl.ds(g * L, L)]

        acc = lax.fori_loop(0, NUM_FIELDS, add_field, bias_vec)
        out_v[pl.ds(g * L, L)] = acc
        return carry

    lax.fori_loop(0, GROUPS, reduce_group, 0)

    pltpu.sync_copy(out_v, out_hbm.at[pl.ds(row_base, CHUNK)])


def kernel(x, table, bias):
    x_perm = x.reshape(NW, CHUNK, NUM_FIELDS).transpose(0, 2, 1)
    table_flat = table[:, 0]
    return _sc_linear(x_perm, table_flat, bias)


# speedup vs baseline: 1.5161x; 1.0939x over previous
import functools

import jax
import jax.numpy as jnp
from jax import lax
from jax.experimental import pallas as pl
from jax.experimental.pallas import tpu as pltpu
from jax.experimental.pallas import tpu_sc as plsc

BATCH = 4096
NUM_FIELDS = 26
NUM_FEATURES = 100000

NC = 2
NS = 16
L = 16
NW = NC * NS
CHUNK = BATCH // NW
GROUPS = CHUNK // L


@functools.partial(
    pl.kernel,
    out_type=jax.ShapeDtypeStruct((BATCH,), jnp.float32),
    mesh=plsc.VectorSubcoreMesh(core_axis_name="c", subcore_axis_name="s"),
    scratch_types=[
        pltpu.VMEM((NUM_FIELDS, CHUNK), jnp.int32),
        pltpu.VMEM((NUM_FIELDS, CHUNK), jnp.float32),
        pltpu.VMEM((L,), jnp.int32),
        pltpu.VMEM((L,), jnp.float32),
        pltpu.VMEM((CHUNK,), jnp.float32),
        pltpu.VMEM_SHARED((NUM_FEATURES,), jnp.float32),
        pltpu.SemaphoreType.DMA,
        pltpu.SemaphoreType.DMA,
    ],
)
def _sc_linear(x_hbm, table_hbm, bias_hbm, out_hbm,
               idx_v, vals_v, zi_v, bias_v, out_v, tbl_sh, gsem, bsem):
    wid = lax.axis_index("s") * NC + lax.axis_index("c")
    sid = lax.axis_index("s")
    row_base = wid * CHUNK

    @pl.when(sid == 0)
    def _():
        pltpu.sync_copy(table_hbm, tbl_sh)

    pltpu.sync_copy(x_hbm.at[:, pl.ds(row_base, CHUNK)], idx_v)

    zi_v[...] = jnp.zeros((L,), jnp.int32)
    bias_cp = pltpu.async_copy(bias_hbm.at[zi_v], bias_v, bsem)

    plsc.subcore_barrier()

    def fire(f, carry):
        pltpu.async_copy(tbl_sh.at[idx_v.at[f]], vals_v.at[f], gsem)
        return carry

    def drain(f, carry):
        pltpu.make_async_copy(tbl_sh.at[idx_v.at[f]], vals_v.at[f], gsem).wait()
        return carry

    lax.fori_loop(0, NUM_FIELDS, fire, 0)
    bias_cp.wait()
    lax.fori_loop(0, NUM_FIELDS, drain, 0)

    bias_vec = bias_v[...]

    def reduce_group(g, carry):
        def add_field(f, acc):
            return acc + vals_v[f, pl.ds(g * L, L)]

        acc = lax.fori_loop(0, NUM_FIELDS, add_field, bias_vec)
        out_v[pl.ds(g * L, L)] = acc
        return carry

    lax.fori_loop(0, GROUPS, reduce_group, 0)

    pltpu.sync_copy(out_v, out_hbm.at[pl.ds(row_base, CHUNK)])


def kernel(x, table, bias):
    return _sc_linear(x.T, table[:, 0], bias)


# speedup vs baseline: 1.5178x; 1.0011x over previous
import functools

import jax
import jax.numpy as jnp
from jax import lax
from jax.experimental import pallas as pl
from jax.experimental.pallas import tpu as pltpu
from jax.experimental.pallas import tpu_sc as plsc

BATCH = 4096
NUM_FIELDS = 26
NUM_FEATURES = 100000

NC = 2
NS = 16
L = 16
NW = NC * NS
CHUNK = BATCH // NW
GROUPS = CHUNK // L


@functools.partial(
    pl.kernel,
    out_type=jax.ShapeDtypeStruct((BATCH,), jnp.float32),
    mesh=plsc.VectorSubcoreMesh(core_axis_name="c", subcore_axis_name="s"),
    scratch_types=[
        pltpu.VMEM((NUM_FIELDS, CHUNK), jnp.int32),
        pltpu.VMEM((NUM_FIELDS, CHUNK), jnp.float32),
        pltpu.VMEM((L,), jnp.int32),
        pltpu.VMEM((L,), jnp.float32),
        pltpu.VMEM((CHUNK,), jnp.float32),
        pltpu.VMEM_SHARED((NUM_FEATURES,), jnp.float32),
        pltpu.SemaphoreType.DMA,
        pltpu.SemaphoreType.DMA,
    ],
)
def _sc_linear(x_hbm, table_hbm, bias_hbm, out_hbm,
               idx_v, vals_v, zi_v, bias_v, out_v, tbl_sh, gsem, bsem):
    wid = lax.axis_index("s") * NC + lax.axis_index("c")
    sid = lax.axis_index("s")
    row_base = wid * CHUNK

    @pl.when(sid == 0)
    def _():
        pltpu.sync_copy(table_hbm, tbl_sh)

    pltpu.sync_copy(x_hbm.at[:, pl.ds(row_base, CHUNK)], idx_v)

    zi_v[...] = jnp.zeros((L,), jnp.int32)
    bias_cp = pltpu.async_copy(bias_hbm.at[zi_v], bias_v, bsem)

    plsc.subcore_barrier()

    def fire(f, carry):
        pltpu.async_copy(tbl_sh.at[idx_v.at[f]], vals_v.at[f], gsem)
        return carry

    def drain(f, carry):
        pltpu.make_async_copy(tbl_sh.at[idx_v.at[f]], vals_v.at[f], gsem).wait()
        return carry

    lax.fori_loop(0, NUM_FIELDS, fire, 0)
    bias_cp.wait()
    lax.fori_loop(0, NUM_FIELDS, drain, 0)

    bias_vec = bias_v[...]

    def reduce_group(g, carry):
        def add_field(f, acc):
            return acc + vals_v[f, pl.ds(g * L, L)]

        acc = lax.fori_loop(0, NUM_FIELDS, add_field, bias_vec)
        out_v[pl.ds(g * L, L)] = acc
        return carry

    lax.fori_loop(0, GROUPS, reduce_group, 0)

    pltpu.sync_copy(out_v, out_hbm.at[pl.ds(row_base, CHUNK)])


def kernel(x, table, bias):
    return _sc_linear(x.T, table.T.reshape(NUM_FEATURES), bias)
